# Initial kernel scaffold; baseline (speedup 1.0000x reference)
import jax
import jax.numpy as jnp
from jax.experimental import pallas as pl


def kernel(ents_embed_input, rels_embed_input, W_ent, W_rel, bias_vec, near_rels_num, edge_index, rel_rows, rel_vals):
    raise NotImplementedError("write your pallas kernel here")


# speedup vs baseline: 3.1390x; 3.1390x over previous
import functools

import jax
import jax.numpy as jnp
from jax import lax
from jax.experimental import pallas as pl
from jax.experimental.pallas import tpu as pltpu
from jax.experimental.pallas import tpu_sc as plsc

MIN_NORM = 1e-10
PROJ_EPS = 1e-5
COMBINE_RELS_WEIGHT = 0.1

N = 10000
D = 128
E = 320000
RE = 200000
NC, NS, L = 2, 16, 16
NW = NC * NS

ACC_ROWS = 10240
ROWS_PER_TILE = ACC_ROWS // NS
ZCH = 32

EPT = E // NW
CH = 80
NCH = EPT // CH

REPAD = NW * 6400
REPT = REPAD // NW
RNCH = REPT // CH

BLK = 1000


def _embed_body(x_ref, w_ref, m_ref):
    x = x_ref[...]
    n = jnp.sqrt(jnp.sum(x * x, axis=1, keepdims=True))
    n = jnp.maximum(n, MIN_NORM)
    n_c = jnp.clip(n, MIN_NORM, 1.0 - PROJ_EPS)
    at = 0.5 * (jnp.log(1.0 + n_c) - jnp.log(1.0 - n_c))
    t = at * x / n
    m_ref[...] = jnp.dot(t, w_ref[...], precision=lax.Precision.HIGHEST,
                         preferred_element_type=jnp.float32)


def _embed(ents, w_ent):
    return pl.pallas_call(
        _embed_body,
        grid=(N // BLK,),
        in_specs=[
            pl.BlockSpec((BLK, D), lambda i: (i, 0)),
            pl.BlockSpec((D, D), lambda i: (0, 0)),
        ],
        out_specs=pl.BlockSpec((BLK, D), lambda i: (i, 0)),
        out_shape=jax.ShapeDtypeStruct((N, D), jnp.float32),
    )(ents, w_ent)


def _sc_edges_body(m_hbm, rows_hbm, cols_hbm, acc_out, den_out,
                   idxr_v, idxc_v, mr_v, mc_v, w_v, zrow_v, zden_v,
                   sem1, sem2, acc_sh, den_sh):
    cid = lax.axis_index("c")
    sid = lax.axis_index("s")
    wid = cid * NS + sid
    zv = jnp.zeros((L,), jnp.float32)

    def _zrow(i, _):
        for k in range(D // L):
            zrow_v[i, pl.ds(k * L, L)] = zv
        return ()
    lax.fori_loop(0, ZCH, _zrow, ())

    def _zdv(i, _):
        zden_v[pl.ds(i * L, L)] = zv
        return ()
    lax.fori_loop(0, ROWS_PER_TILE // L, _zdv, ())

    def _zacc(j, _):
        pltpu.sync_copy(zrow_v, acc_sh.at[pl.ds(sid * ROWS_PER_TILE + j * ZCH, ZCH)])
        return ()
    lax.fori_loop(0, ROWS_PER_TILE // ZCH, _zacc, ())

    pltpu.sync_copy(zden_v, den_sh.at[pl.ds(sid * ROWS_PER_TILE, ROWS_PER_TILE)])

    plsc.subcore_barrier()

    base = wid * EPT

    def _chunk(i, _):
        off = base + i * CH
        pltpu.sync_copy(rows_hbm.at[pl.ds(off, CH)], idxr_v)
        pltpu.sync_copy(cols_hbm.at[pl.ds(off, CH)], idxc_v)
        c1 = pltpu.async_copy(m_hbm.at[idxr_v], mr_v, sem1)
        c2 = pltpu.async_copy(m_hbm.at[idxc_v], mc_v, sem2)
        c1.wait()
        c2.wait()

        iot = lax.iota(jnp.int32, L)

        def _group(g, _):
            rowv = g * L + iot
            def _dd(d, s):
                dv = jnp.full((L,), d, jnp.int32)
                a = plsc.load_gather(mr_v, [rowv, dv])
                b = plsc.load_gather(mc_v, [rowv, dv])
                return s + a * b
            s = lax.fori_loop(0, D, _dd, jnp.zeros((L,), jnp.float32))
            w16 = jnp.exp(s)
            w_v[pl.ds(g * L, L)] = w16
            for j in range(L):
                wv = jnp.broadcast_to(w16[j], (L,))
                e = g * L + j
                for k in range(D // L):
                    mc_v[e, pl.ds(k * L, L)] = mc_v[e, pl.ds(k * L, L)] * wv
            return ()
        lax.fori_loop(0, CH // L, _group, ())

        pltpu.sync_copy(mc_v, acc_sh.at[idxr_v], add=True)
        pltpu.sync_copy(w_v, den_sh.at[idxr_v], add=True)
        return ()
    lax.fori_loop(0, NCH, _chunk, ())

    plsc.subcore_barrier()
    pltpu.sync_copy(acc_sh.at[pl.ds(sid * ROWS_PER_TILE, ROWS_PER_TILE)],
                    acc_out.at[cid, pl.ds(sid * ROWS_PER_TILE, ROWS_PER_TILE)])
    pltpu.sync_copy(den_sh.at[pl.ds(sid * ROWS_PER_TILE, ROWS_PER_TILE)],
                    den_out.at[cid, pl.ds(sid * ROWS_PER_TILE, ROWS_PER_TILE)])


_sc_edges = functools.partial(
    pl.kernel,
    out_type=(jax.ShapeDtypeStruct((NC, ACC_ROWS, D), jnp.float32),
              jax.ShapeDtypeStruct((NC, ACC_ROWS), jnp.float32)),
    mesh=plsc.VectorSubcoreMesh(core_axis_name="c", subcore_axis_name="s"),
    compiler_params=pltpu.CompilerParams(needs_layout_passes=False),
    scratch_types=[
        pltpu.VMEM((CH,), jnp.int32),
        pltpu.VMEM((CH,), jnp.int32),
        pltpu.VMEM((CH, D), jnp.float32),
        pltpu.VMEM((CH, D), jnp.float32),
        pltpu.VMEM((CH,), jnp.float32),
        pltpu.VMEM((ZCH, D), jnp.float32),
        pltpu.VMEM((ROWS_PER_TILE,), jnp.float32),
        pltpu.SemaphoreType.DMA,
        pltpu.SemaphoreType.DMA,
        pltpu.VMEM_SHARED((ACC_ROWS, D), jnp.float32),
        pltpu.VMEM_SHARED((ACC_ROWS,), jnp.float32),
    ],
)(_sc_edges_body)


def _sc_rels_body(rels_hbm, rrows_hbm, rvals_hbm, acc_out,
                  row_v, val_v, emb_v, zrow_v, sem1, acc_sh):
    cid = lax.axis_index("c")
    sid = lax.axis_index("s")
    wid = cid * NS + sid
    zv = jnp.zeros((L,), jnp.float32)

    def _zrow(i, _):
        for k in range(D // L):
            zrow_v[i, pl.ds(k * L, L)] = zv
        return ()
    lax.fori_loop(0, ZCH, _zrow, ())

    def _zacc(j, _):
        pltpu.sync_copy(zrow_v, acc_sh.at[pl.ds(sid * ROWS_PER_TILE + j * ZCH, ZCH)])
        return ()
    lax.fori_loop(0, ROWS_PER_TILE // ZCH, _zacc, ())

    plsc.subcore_barrier()

    base = wid * REPT

    def _chunk(i, _):
        off = base + i * CH
        pltpu.sync_copy(rrows_hbm.at[pl.ds(off, CH)], row_v)
        pltpu.sync_copy(rvals_hbm.at[pl.ds(off, CH)], val_v)
        pltpu.async_copy(rels_hbm.at[val_v], emb_v, sem1).wait()
        pltpu.sync_copy(emb_v, acc_sh.at[row_v], add=True)
        return ()
    lax.fori_loop(0, RNCH, _chunk, ())

    plsc.subcore_barrier()
    pltpu.sync_copy(acc_sh.at[pl.ds(sid * ROWS_PER_TILE, ROWS_PER_TILE)],
                    acc_out.at[cid, pl.ds(sid * ROWS_PER_TILE, ROWS_PER_TILE)])


_sc_rels = functools.partial(
    pl.kernel,
    out_type=jax.ShapeDtypeStruct((NC, ACC_ROWS, D), jnp.float32),
    mesh=plsc.VectorSubcoreMesh(core_axis_name="c", subcore_axis_name="s"),
    compiler_params=pltpu.CompilerParams(needs_layout_passes=False),
    scratch_types=[
        pltpu.VMEM((CH,), jnp.int32),
        pltpu.VMEM((CH,), jnp.int32),
        pltpu.VMEM((CH, D), jnp.float32),
        pltpu.VMEM((ZCH, D), jnp.float32),
        pltpu.SemaphoreType.DMA,
        pltpu.VMEM_SHARED((ACC_ROWS, D), jnp.float32),
    ],
)(_sc_rels_body)


def _norm(x):
    return jnp.maximum(jnp.sqrt(jnp.sum(x * x, axis=-1, keepdims=True)), MIN_NORM)


def _exp_map_zero(v):
    n = _norm(v)
    return jnp.tanh(n) * v / n


def _proj(x):
    n = _norm(x)
    max_norm = 1.0 - PROJ_EPS
    return jnp.where(n > max_norm, x / n * max_norm, x)


def _mobius(x, y):
    x2 = jnp.sum(x * x, axis=-1, keepdims=True)
    y2 = jnp.sum(y * y, axis=-1, keepdims=True)
    xy = jnp.sum(x * y, axis=-1, keepdims=True)
    num = (1.0 + 2.0 * xy + y2) * x + (1.0 - x2) * y
    den = 1.0 + 2.0 * xy + x2 * y2
    return num / jnp.clip(den, MIN_NORM, None)


def _final_body(acc_ref, den_ref, acc2_ref, num_ref, bias_ref, out_ref):
    acc = acc_ref[0] + acc_ref[1]
    a2 = acc2_ref[0] + acc2_ref[1]
    den = jnp.clip(den_ref[...], 1e-15, None)
    v = acc / den + COMBINE_RELS_WEIGHT * (a2 / num_ref[...])
    out1 = _proj(_exp_map_zero(v))
    b = _proj(_exp_map_zero(bias_ref[...]))
    out_ref[...] = _proj(_mobius(out1, b))


def _final(acc_p, den, acc2_p, num, bias_vec):
    return pl.pallas_call(
        _final_body,
        grid=(N // BLK,),
        in_specs=[
            pl.BlockSpec((NC, BLK, D), lambda i: (0, i, 0)),
            pl.BlockSpec((BLK, 1), lambda i: (i, 0)),
            pl.BlockSpec((NC, BLK, D), lambda i: (0, i, 0)),
            pl.BlockSpec((BLK, 1), lambda i: (i, 0)),
            pl.BlockSpec((1, D), lambda i: (0, 0)),
        ],
        out_specs=pl.BlockSpec((BLK, D), lambda i: (i, 0)),
        out_shape=jax.ShapeDtypeStruct((N, D), jnp.float32),
    )(acc_p, den, acc2_p, num, bias_vec)


def kernel(ents_embed_input, rels_embed_input, W_ent, W_rel, bias_vec,
           near_rels_num, edge_index, rel_rows, rel_vals):
    m = _embed(ents_embed_input, W_ent)
    rows = edge_index[0]
    cols = edge_index[1]
    acc_p, den_p = _sc_edges(m, rows, cols)

    pad = REPAD - RE
    rrows = jnp.concatenate([rel_rows, jnp.full((pad,), N, jnp.int32)])
    rvals = jnp.concatenate([rel_vals, jnp.zeros((pad,), jnp.int32)])
    acc2_p = _sc_rels(rels_embed_input, rrows, rvals)

    den = (den_p[0, :N] + den_p[1, :N]).reshape(N, 1)
    num = near_rels_num.reshape(N, 1)
    return _final(acc_p[:, :N], den, acc2_p[:, :N], num, bias_vec)


# speedup vs baseline: 3.9144x; 1.2470x over previous
import functools

import jax
import jax.numpy as jnp
from jax import lax
from jax.experimental import pallas as pl
from jax.experimental.pallas import tpu as pltpu
from jax.experimental.pallas import tpu_sc as plsc

MIN_NORM = 1e-10
PROJ_EPS = 1e-5
COMBINE_RELS_WEIGHT = 0.1

N = 10000
D = 128
E = 320000
RE = 200000
NC, NS, L = 2, 16, 16
NW = NC * NS

ACC_ROWS = 10240
ROWS_PER_TILE = ACC_ROWS // NS
ZCH = 16

EPT = E // NW
CH = 16
NCH = EPT // CH
NBUF = 4
NBLK = 156

REPAD = NW * 6400
REPT = REPAD // NW
RNCH = REPT // CH
RBUF = 8
RBLK = RNCH // RBUF


def _embed_body(x_ref, w_ref, m_ref):
    x = x_ref[...]
    n = jnp.sqrt(jnp.sum(x * x, axis=1, keepdims=True))
    n = jnp.maximum(n, MIN_NORM)
    n_c = jnp.clip(n, MIN_NORM, 1.0 - PROJ_EPS)
    at = 0.5 * (jnp.log(1.0 + n_c) - jnp.log(1.0 - n_c))
    t = at * x / n
    m_ref[...] = jnp.dot(t, w_ref[...], precision=lax.Precision.HIGHEST,
                         preferred_element_type=jnp.float32)


BLK = 1000


def _embed(ents, w_ent):
    return pl.pallas_call(
        _embed_body,
        grid=(N // BLK,),
        in_specs=[
            pl.BlockSpec((BLK, D), lambda i: (i, 0)),
            pl.BlockSpec((D, D), lambda i: (0, 0)),
        ],
        out_specs=pl.BlockSpec((BLK, D), lambda i: (i, 0)),
        out_shape=jax.ShapeDtypeStruct((N, D), jnp.float32),
    )(ents, w_ent)


def _sc_edges_body(m_hbm, rows_hbm, cols_hbm, acc_out, den_out,
                   idxr1_v, idxc1_v, mr_v, mc_v, w_v, zrow_v, zden_v,
                   gs0, gs1, gs2, gs3, ss0, ss1, ss2, ss3, acc_sh, den_sh):
    cid = lax.axis_index("c")
    sid = lax.axis_index("s")
    wid = cid * NS + sid
    zv = jnp.zeros((L,), jnp.float32)
    gsems = [gs0, gs1, gs2, gs3]
    ssems = [ss0, ss1, ss2, ss3]

    def _zrow(i, _):
        for k in range(D // L):
            zrow_v[i, pl.ds(k * L, L)] = zv
        return ()
    lax.fori_loop(0, ZCH, _zrow, ())

    def _zdv(i, _):
        zden_v[pl.ds(i * L, L)] = zv
        return ()
    lax.fori_loop(0, ROWS_PER_TILE // L, _zdv, ())

    def _zacc(j, _):
        pltpu.sync_copy(zrow_v, acc_sh.at[pl.ds(sid * ROWS_PER_TILE + j * ZCH, ZCH)])
        return ()
    lax.fori_loop(0, ROWS_PER_TILE // ZCH, _zacc, ())

    pltpu.sync_copy(zden_v, den_sh.at[pl.ds(sid * ROWS_PER_TILE, ROWS_PER_TILE)])

    pltpu.sync_copy(rows_hbm.at[wid], idxr1_v)
    pltpu.sync_copy(cols_hbm.at[wid], idxc1_v)

    iot = lax.iota(jnp.int32, L)

    def _ridx(i):
        return plsc.load_gather(idxr1_v, [i * CH + iot])

    def _cidx(i):
        return plsc.load_gather(idxc1_v, [i * CH + iot])

    def _fire_gather(i, b):
        pltpu.async_copy(m_hbm.at[_ridx(i)], mr_v.at[b], gsems[b])
        pltpu.async_copy(m_hbm.at[_cidx(i)], mc_v.at[b], gsems[b])

    def _wait_gather(i, b):
        pltpu.make_async_copy(m_hbm.at[_ridx(i)], mr_v.at[b], gsems[b]).wait()
        pltpu.make_async_copy(m_hbm.at[_cidx(i)], mc_v.at[b], gsems[b]).wait()

    def _fire_scatter(i, b):
        rv = _ridx(i)
        pltpu.async_copy(mc_v.at[b], acc_sh.at[rv], ssems[b], add=True)
        pltpu.async_copy(w_v.at[b], den_sh.at[rv], ssems[b], add=True)

    def _wait_scatter(i, b):
        rv = _ridx(i)
        pltpu.make_async_copy(mc_v.at[b], acc_sh.at[rv], ssems[b]).wait()
        pltpu.make_async_copy(w_v.at[b], den_sh.at[rv], ssems[b]).wait()

    def _compute(b):
        def _dd(d, s):
            dv = jnp.full((L,), d, jnp.int32)
            a = plsc.load_gather(mr_v.at[b], [iot, dv])
            c = plsc.load_gather(mc_v.at[b], [iot, dv])
            return s + a * c
        s = lax.fori_loop(0, D, _dd, jnp.zeros((L,), jnp.float32), unroll=8)
        w16 = jnp.exp(s)
        w_v[b, pl.ds(0, L)] = w16
        for j in range(L):
            wv = jnp.broadcast_to(w16[j], (L,))
            for k in range(D // L):
                mc_v[b, j, pl.ds(k * L, L)] = mc_v[b, j, pl.ds(k * L, L)] * wv

    plsc.subcore_barrier()

    for b in range(NBUF):
        _fire_gather(b, b)

    def _blk(g, _):
        for b in range(NBUF):
            i = g * NBUF + b
            _wait_gather(i, b)

            @pl.when(g >= 1)
            def _():
                _wait_scatter(i - NBUF, b)

            _compute(b)

            @pl.when(g < NBLK - 1)
            def _():
                _fire_gather(i + NBUF, b)

            _fire_scatter(i, b)
        return ()
    lax.fori_loop(0, NBLK, _blk, ())

    for b in range(NBUF):
        _wait_scatter((NBLK - 1) * NBUF + b, b)

    for t in range(NBUF * NBLK, NCH):
        b = t - NBUF * NBLK
        pltpu.async_copy(m_hbm.at[_ridx(t)], mr_v.at[b], gsems[b]).wait()
        pltpu.async_copy(m_hbm.at[_cidx(t)], mc_v.at[b], gsems[b]).wait()
        _compute(b)
        rv = _ridx(t)
        pltpu.async_copy(mc_v.at[b], acc_sh.at[rv], ssems[b], add=True).wait()
        pltpu.async_copy(w_v.at[b], den_sh.at[rv], ssems[b], add=True).wait()

    plsc.subcore_barrier()
    pltpu.sync_copy(acc_sh.at[pl.ds(sid * ROWS_PER_TILE, ROWS_PER_TILE)],
                    acc_out.at[cid, pl.ds(sid * ROWS_PER_TILE, ROWS_PER_TILE)])
    pltpu.sync_copy(den_sh.at[pl.ds(sid * ROWS_PER_TILE, ROWS_PER_TILE)],
                    den_out.at[cid, pl.ds(sid * ROWS_PER_TILE, ROWS_PER_TILE)])


_sc_edges = functools.partial(
    pl.kernel,
    out_type=(jax.ShapeDtypeStruct((NC, ACC_ROWS, D), jnp.float32),
              jax.ShapeDtypeStruct((NC, ACC_ROWS), jnp.float32)),
    mesh=plsc.VectorSubcoreMesh(core_axis_name="c", subcore_axis_name="s"),
    compiler_params=pltpu.CompilerParams(needs_layout_passes=False),
    scratch_types=(
        [
            pltpu.VMEM((EPT,), jnp.int32),
            pltpu.VMEM((EPT,), jnp.int32),
            pltpu.VMEM((NBUF, CH, D), jnp.float32),
            pltpu.VMEM((NBUF, CH, D), jnp.float32),
            pltpu.VMEM((NBUF, CH), jnp.float32),
            pltpu.VMEM((ZCH, D), jnp.float32),
            pltpu.VMEM((ROWS_PER_TILE,), jnp.float32),
        ]
        + [pltpu.SemaphoreType.DMA] * (2 * NBUF)
        + [
            pltpu.VMEM_SHARED((ACC_ROWS, D), jnp.float32),
            pltpu.VMEM_SHARED((ACC_ROWS,), jnp.float32),
        ]
    ),
)(_sc_edges_body)


def _sc_rels_body(rels_hbm, rrows_hbm, rvals_hbm, acc_out,
                  row1_v, val1_v, emb_v, zrow_v,
                  g0, g1, g2, g3, g4, g5, g6, g7,
                  s0, s1, s2, s3, s4, s5, s6, s7, acc_sh):
    cid = lax.axis_index("c")
    sid = lax.axis_index("s")
    wid = cid * NS + sid
    zv = jnp.zeros((L,), jnp.float32)
    gsems = [g0, g1, g2, g3, g4, g5, g6, g7]
    ssems = [s0, s1, s2, s3, s4, s5, s6, s7]

    def _zrow(i, _):
        for k in range(D // L):
            zrow_v[i, pl.ds(k * L, L)] = zv
        return ()
    lax.fori_loop(0, ZCH, _zrow, ())

    def _zacc(j, _):
        pltpu.sync_copy(zrow_v, acc_sh.at[pl.ds(sid * ROWS_PER_TILE + j * ZCH, ZCH)])
        return ()
    lax.fori_loop(0, ROWS_PER_TILE // ZCH, _zacc, ())

    pltpu.sync_copy(rrows_hbm.at[wid], row1_v)
    pltpu.sync_copy(rvals_hbm.at[wid], val1_v)

    iot = lax.iota(jnp.int32, L)

    def _rowi(i):
        return plsc.load_gather(row1_v, [i * CH + iot])

    def _vali(i):
        return plsc.load_gather(val1_v, [i * CH + iot])

    plsc.subcore_barrier()

    def _blk(g, _):
        for b in range(RBUF):
            i = g * RBUF + b

            @pl.when(g >= 1)
            def _():
                pltpu.make_async_copy(emb_v.at[b],
                                      acc_sh.at[_rowi(i - RBUF)],
                                      ssems[b]).wait()

            pltpu.async_copy(rels_hbm.at[_vali(i)], emb_v.at[b], gsems[b])
        for b in range(RBUF):
            i = g * RBUF + b
            pltpu.make_async_copy(rels_hbm.at[_vali(i)], emb_v.at[b],
                                  gsems[b]).wait()
            pltpu.async_copy(emb_v.at[b], acc_sh.at[_rowi(i)], ssems[b],
                             add=True)
        return ()
    lax.fori_loop(0, RBLK, _blk, ())

    for b in range(RBUF):
        pltpu.make_async_copy(emb_v.at[b],
                              acc_sh.at[_rowi((RBLK - 1) * RBUF + b)],
                              ssems[b]).wait()

    plsc.subcore_barrier()
    pltpu.sync_copy(acc_sh.at[pl.ds(sid * ROWS_PER_TILE, ROWS_PER_TILE)],
                    acc_out.at[cid, pl.ds(sid * ROWS_PER_TILE, ROWS_PER_TILE)])


_sc_rels = functools.partial(
    pl.kernel,
    out_type=jax.ShapeDtypeStruct((NC, ACC_ROWS, D), jnp.float32),
    mesh=plsc.VectorSubcoreMesh(core_axis_name="c", subcore_axis_name="s"),
    compiler_params=pltpu.CompilerParams(needs_layout_passes=False),
    scratch_types=(
        [
            pltpu.VMEM((REPT,), jnp.int32),
            pltpu.VMEM((REPT,), jnp.int32),
            pltpu.VMEM((RBUF, CH, D), jnp.float32),
            pltpu.VMEM((ZCH, D), jnp.float32),
        ]
        + [pltpu.SemaphoreType.DMA] * (2 * RBUF)
        + [pltpu.VMEM_SHARED((ACC_ROWS, D), jnp.float32)]
    ),
)(_sc_rels_body)


def _norm(x):
    return jnp.maximum(jnp.sqrt(jnp.sum(x * x, axis=-1, keepdims=True)), MIN_NORM)


def _exp_map_zero(v):
    n = _norm(v)
    return jnp.tanh(n) * v / n


def _proj(x):
    n = _norm(x)
    max_norm = 1.0 - PROJ_EPS
    return jnp.where(n > max_norm, x / n * max_norm, x)


def _mobius(x, y):
    x2 = jnp.sum(x * x, axis=-1, keepdims=True)
    y2 = jnp.sum(y * y, axis=-1, keepdims=True)
    xy = jnp.sum(x * y, axis=-1, keepdims=True)
    num = (1.0 + 2.0 * xy + y2) * x + (1.0 - x2) * y
    den = 1.0 + 2.0 * xy + x2 * y2
    return num / jnp.clip(den, MIN_NORM, None)


def _final_body(acc_ref, den_ref, acc2_ref, num_ref, bias_ref, out_ref):
    acc = acc_ref[0] + acc_ref[1]
    a2 = acc2_ref[0] + acc2_ref[1]
    den = jnp.clip(den_ref[...], 1e-15, None)
    v = acc / den + COMBINE_RELS_WEIGHT * (a2 / num_ref[...])
    out1 = _proj(_exp_map_zero(v))
    b = _proj(_exp_map_zero(bias_ref[...]))
    out_ref[...] = _proj(_mobius(out1, b))


def _final(acc_p, den, acc2_p, num, bias_vec):
    return pl.pallas_call(
        _final_body,
        grid=(N // BLK,),
        in_specs=[
            pl.BlockSpec((NC, BLK, D), lambda i: (0, i, 0)),
            pl.BlockSpec((BLK, 1), lambda i: (i, 0)),
            pl.BlockSpec((NC, BLK, D), lambda i: (0, i, 0)),
            pl.BlockSpec((BLK, 1), lambda i: (i, 0)),
            pl.BlockSpec((1, D), lambda i: (0, 0)),
        ],
        out_specs=pl.BlockSpec((BLK, D), lambda i: (i, 0)),
        out_shape=jax.ShapeDtypeStruct((N, D), jnp.float32),
    )(acc_p, den, acc2_p, num, bias_vec)


def kernel(ents_embed_input, rels_embed_input, W_ent, W_rel, bias_vec,
           near_rels_num, edge_index, rel_rows, rel_vals):
    m = _embed(ents_embed_input, W_ent)
    rows = edge_index[0].reshape(NW, EPT)
    cols = edge_index[1].reshape(NW, EPT)
    acc_p, den_p = _sc_edges(m, rows, cols)

    pad = REPAD - RE
    rrows = jnp.concatenate(
        [rel_rows, jnp.full((pad,), N, jnp.int32)]).reshape(NW, REPT)
    rvals = jnp.concatenate(
        [rel_vals, jnp.zeros((pad,), jnp.int32)]).reshape(NW, REPT)
    acc2_p = _sc_rels(rels_embed_input, rrows, rvals)

    den = (den_p[0, :N] + den_p[1, :N]).reshape(N, 1)
    num = near_rels_num.reshape(N, 1)
    return _final(acc_p[:, :N], den, acc2_p[:, :N], num, bias_vec)


# speedup vs baseline: 7.7117x; 1.9701x over previous
import functools

import jax
import jax.numpy as jnp
from jax import lax
from jax.experimental import pallas as pl
from jax.experimental.pallas import tpu as pltpu
from jax.experimental.pallas import tpu_sc as plsc

MIN_NORM = 1e-10
PROJ_EPS = 1e-5
COMBINE_RELS_WEIGHT = 0.1

N = 10000
D = 128
E = 320000
RE = 200000
NC, NS, L = 2, 16, 16
NW = NC * NS

ACC_ROWS = 10240
ROWS_PER_TILE = ACC_ROWS // NS
ZCH = 16

EPT = E // NW
CH = 16
NCH = EPT // CH
NBUF = 4
NBLK = 156

REPAD = NW * 6400
REPT = REPAD // NW
RNCH = REPT // CH
RBUF = 8
RBLK = RNCH // RBUF


def _embed_body(x_ref, w_ref, m_ref):
    x = x_ref[...]
    n = jnp.sqrt(jnp.sum(x * x, axis=1, keepdims=True))
    n = jnp.maximum(n, MIN_NORM)
    n_c = jnp.clip(n, MIN_NORM, 1.0 - PROJ_EPS)
    at = 0.5 * (jnp.log(1.0 + n_c) - jnp.log(1.0 - n_c))
    t = at * x / n
    m_ref[...] = jnp.dot(t, w_ref[...], precision=lax.Precision.HIGHEST,
                         preferred_element_type=jnp.float32)


BLK = 1000


def _embed(ents, w_ent):
    return pl.pallas_call(
        _embed_body,
        grid=(N // BLK,),
        in_specs=[
            pl.BlockSpec((BLK, D), lambda i: (i, 0)),
            pl.BlockSpec((D, D), lambda i: (0, 0)),
        ],
        out_specs=pl.BlockSpec((BLK, D), lambda i: (i, 0)),
        out_shape=jax.ShapeDtypeStruct((N, D), jnp.float32),
    )(ents, w_ent)


def _sc_edges_body(m_hbm, rows_hbm, cols_hbm, acc_out, den_out,
                   idxr1_v, idxc1_v, mr_v, mc_v, w_v, dot_v, zrow_v, zden_v,
                   gs0, gs1, gs2, gs3, ss0, ss1, ss2, ss3, acc_sh, den_sh):
    cid = lax.axis_index("c")
    sid = lax.axis_index("s")
    wid = cid * NS + sid
    zv = jnp.zeros((L,), jnp.float32)
    gsems = [gs0, gs1, gs2, gs3]
    ssems = [ss0, ss1, ss2, ss3]

    def _zrow(i, _):
        for k in range(D // L):
            zrow_v[i, pl.ds(k * L, L)] = zv
        return ()
    lax.fori_loop(0, ZCH, _zrow, ())

    def _zdv(i, _):
        zden_v[pl.ds(i * L, L)] = zv
        return ()
    lax.fori_loop(0, ROWS_PER_TILE // L, _zdv, ())

    def _zacc(j, _):
        pltpu.sync_copy(zrow_v, acc_sh.at[pl.ds(sid * ROWS_PER_TILE + j * ZCH, ZCH)])
        return ()
    lax.fori_loop(0, ROWS_PER_TILE // ZCH, _zacc, ())

    pltpu.sync_copy(zden_v, den_sh.at[pl.ds(sid * ROWS_PER_TILE, ROWS_PER_TILE)])

    pltpu.sync_copy(rows_hbm.at[wid], idxr1_v)
    pltpu.sync_copy(cols_hbm.at[wid], idxc1_v)

    iot = lax.iota(jnp.int32, L)

    def _ridx(i):
        return plsc.load_gather(idxr1_v, [i * CH + iot])

    def _cidx(i):
        return plsc.load_gather(idxc1_v, [i * CH + iot])

    def _fire_gather(i, b):
        pltpu.async_copy(m_hbm.at[_ridx(i)], mr_v.at[b], gsems[b])
        pltpu.async_copy(m_hbm.at[_cidx(i)], mc_v.at[b], gsems[b])

    def _wait_gather(i, b):
        pltpu.make_async_copy(m_hbm.at[_ridx(i)], mr_v.at[b], gsems[b]).wait()
        pltpu.make_async_copy(m_hbm.at[_cidx(i)], mc_v.at[b], gsems[b]).wait()

    def _fire_scatter(i, b):
        rv = _ridx(i)
        pltpu.async_copy(mc_v.at[b], acc_sh.at[rv], ssems[b], add=True)
        pltpu.async_copy(w_v.at[b], den_sh.at[rv], ssems[b], add=True)

    def _wait_scatter(i, b):
        rv = _ridx(i)
        pltpu.make_async_copy(mc_v.at[b], acc_sh.at[rv], ssems[b]).wait()
        pltpu.make_async_copy(w_v.at[b], den_sh.at[rv], ssems[b]).wait()

    def _compute(b):
        for j in range(L):
            acc = mr_v[b, j, pl.ds(0, L)] * mc_v[b, j, pl.ds(0, L)]
            for k in range(1, D // L):
                acc = acc + (mr_v[b, j, pl.ds(k * L, L)]
                             * mc_v[b, j, pl.ds(k * L, L)])
            plsc.store_scatter(dot_v, [j * (L + 1) + iot], acc)
        s = plsc.load_gather(dot_v, [iot * (L + 1)])
        for k in range(1, L):
            s = s + plsc.load_gather(dot_v, [iot * (L + 1) + k])
        w16 = jnp.exp(s)
        w_v[b, pl.ds(0, L)] = w16
        for j in range(L):
            wv = jnp.broadcast_to(w16[j], (L,))
            for k in range(D // L):
                mc_v[b, j, pl.ds(k * L, L)] = mc_v[b, j, pl.ds(k * L, L)] * wv

    plsc.subcore_barrier()

    for b in range(NBUF):
        _fire_gather(b, b)

    def _blk(g, _):
        for b in range(NBUF):
            i = g * NBUF + b
            _wait_gather(i, b)

            @pl.when(g >= 1)
            def _():
                _wait_scatter(i - NBUF, b)

            _compute(b)

            @pl.when(g < NBLK - 1)
            def _():
                _fire_gather(i + NBUF, b)

            _fire_scatter(i, b)
        return ()
    lax.fori_loop(0, NBLK, _blk, ())

    for b in range(NBUF):
        _wait_scatter((NBLK - 1) * NBUF + b, b)

    for t in range(NBUF * NBLK, NCH):
        b = t - NBUF * NBLK
        pltpu.async_copy(m_hbm.at[_ridx(t)], mr_v.at[b], gsems[b]).wait()
        pltpu.async_copy(m_hbm.at[_cidx(t)], mc_v.at[b], gsems[b]).wait()
        _compute(b)
        rv = _ridx(t)
        pltpu.async_copy(mc_v.at[b], acc_sh.at[rv], ssems[b], add=True).wait()
        pltpu.async_copy(w_v.at[b], den_sh.at[rv], ssems[b], add=True).wait()

    plsc.subcore_barrier()
    pltpu.sync_copy(acc_sh.at[pl.ds(sid * ROWS_PER_TILE, ROWS_PER_TILE)],
                    acc_out.at[cid, pl.ds(sid * ROWS_PER_TILE, ROWS_PER_TILE)])
    pltpu.sync_copy(den_sh.at[pl.ds(sid * ROWS_PER_TILE, ROWS_PER_TILE)],
                    den_out.at[cid, pl.ds(sid * ROWS_PER_TILE, ROWS_PER_TILE)])


_sc_edges = functools.partial(
    pl.kernel,
    out_type=(jax.ShapeDtypeStruct((NC, ACC_ROWS, D), jnp.float32),
              jax.ShapeDtypeStruct((NC, ACC_ROWS), jnp.float32)),
    mesh=plsc.VectorSubcoreMesh(core_axis_name="c", subcore_axis_name="s"),
    compiler_params=pltpu.CompilerParams(needs_layout_passes=False),
    scratch_types=(
        [
            pltpu.VMEM((EPT,), jnp.int32),
            pltpu.VMEM((EPT,), jnp.int32),
            pltpu.VMEM((NBUF, CH, D), jnp.float32),
            pltpu.VMEM((NBUF, CH, D), jnp.float32),
            pltpu.VMEM((NBUF, CH), jnp.float32),
            pltpu.VMEM((L * (L + 1),), jnp.float32),
            pltpu.VMEM((ZCH, D), jnp.float32),
            pltpu.VMEM((ROWS_PER_TILE,), jnp.float32),
        ]
        + [pltpu.SemaphoreType.DMA] * (2 * NBUF)
        + [
            pltpu.VMEM_SHARED((ACC_ROWS, D), jnp.float32),
            pltpu.VMEM_SHARED((ACC_ROWS,), jnp.float32),
        ]
    ),
)(_sc_edges_body)


def _sc_rels_body(rels_hbm, rrows_hbm, rvals_hbm, acc_out,
                  row1_v, val1_v, emb_v, zrow_v,
                  g0, g1, g2, g3, g4, g5, g6, g7,
                  s0, s1, s2, s3, s4, s5, s6, s7, acc_sh):
    cid = lax.axis_index("c")
    sid = lax.axis_index("s")
    wid = cid * NS + sid
    zv = jnp.zeros((L,), jnp.float32)
    gsems = [g0, g1, g2, g3, g4, g5, g6, g7]
    ssems = [s0, s1, s2, s3, s4, s5, s6, s7]

    def _zrow(i, _):
        for k in range(D // L):
            zrow_v[i, pl.ds(k * L, L)] = zv
        return ()
    lax.fori_loop(0, ZCH, _zrow, ())

    def _zacc(j, _):
        pltpu.sync_copy(zrow_v, acc_sh.at[pl.ds(sid * ROWS_PER_TILE + j * ZCH, ZCH)])
        return ()
    lax.fori_loop(0, ROWS_PER_TILE // ZCH, _zacc, ())

    pltpu.sync_copy(rrows_hbm.at[wid], row1_v)
    pltpu.sync_copy(rvals_hbm.at[wid], val1_v)

    iot = lax.iota(jnp.int32, L)

    def _rowi(i):
        return plsc.load_gather(row1_v, [i * CH + iot])

    def _vali(i):
        return plsc.load_gather(val1_v, [i * CH + iot])

    plsc.subcore_barrier()

    def _blk(g, _):
        for b in range(RBUF):
            i = g * RBUF + b

            @pl.when(g >= 1)
            def _():
                pltpu.make_async_copy(emb_v.at[b],
                                      acc_sh.at[_rowi(i - RBUF)],
                                      ssems[b]).wait()

            pltpu.async_copy(rels_hbm.at[_vali(i)], emb_v.at[b], gsems[b])
        for b in range(RBUF):
            i = g * RBUF + b
            pltpu.make_async_copy(rels_hbm.at[_vali(i)], emb_v.at[b],
                                  gsems[b]).wait()
            pltpu.async_copy(emb_v.at[b], acc_sh.at[_rowi(i)], ssems[b],
                             add=True)
        return ()
    lax.fori_loop(0, RBLK, _blk, ())

    for b in range(RBUF):
        pltpu.make_async_copy(emb_v.at[b],
                              acc_sh.at[_rowi((RBLK - 1) * RBUF + b)],
                              ssems[b]).wait()

    plsc.subcore_barrier()
    pltpu.sync_copy(acc_sh.at[pl.ds(sid * ROWS_PER_TILE, ROWS_PER_TILE)],
                    acc_out.at[cid, pl.ds(sid * ROWS_PER_TILE, ROWS_PER_TILE)])


_sc_rels = functools.partial(
    pl.kernel,
    out_type=jax.ShapeDtypeStruct((NC, ACC_ROWS, D), jnp.float32),
    mesh=plsc.VectorSubcoreMesh(core_axis_name="c", subcore_axis_name="s"),
    compiler_params=pltpu.CompilerParams(needs_layout_passes=False),
    scratch_types=(
        [
            pltpu.VMEM((REPT,), jnp.int32),
            pltpu.VMEM((REPT,), jnp.int32),
            pltpu.VMEM((RBUF, CH, D), jnp.float32),
            pltpu.VMEM((ZCH, D), jnp.float32),
        ]
        + [pltpu.SemaphoreType.DMA] * (2 * RBUF)
        + [pltpu.VMEM_SHARED((ACC_ROWS, D), jnp.float32)]
    ),
)(_sc_rels_body)


def _norm(x):
    return jnp.maximum(jnp.sqrt(jnp.sum(x * x, axis=-1, keepdims=True)), MIN_NORM)


def _exp_map_zero(v):
    n = _norm(v)
    return jnp.tanh(n) * v / n


def _proj(x):
    n = _norm(x)
    max_norm = 1.0 - PROJ_EPS
    return jnp.where(n > max_norm, x / n * max_norm, x)


def _mobius(x, y):
    x2 = jnp.sum(x * x, axis=-1, keepdims=True)
    y2 = jnp.sum(y * y, axis=-1, keepdims=True)
    xy = jnp.sum(x * y, axis=-1, keepdims=True)
    num = (1.0 + 2.0 * xy + y2) * x + (1.0 - x2) * y
    den = 1.0 + 2.0 * xy + x2 * y2
    return num / jnp.clip(den, MIN_NORM, None)


def _final_body(acc_ref, den_ref, acc2_ref, num_ref, bias_ref, out_ref):
    acc = acc_ref[0] + acc_ref[1]
    a2 = acc2_ref[0] + acc2_ref[1]
    den = jnp.clip(den_ref[...], 1e-15, None)
    v = acc / den + COMBINE_RELS_WEIGHT * (a2 / num_ref[...])
    out1 = _proj(_exp_map_zero(v))
    b = _proj(_exp_map_zero(bias_ref[...]))
    out_ref[...] = _proj(_mobius(out1, b))


def _final(acc_p, den, acc2_p, num, bias_vec):
    return pl.pallas_call(
        _final_body,
        grid=(N // BLK,),
        in_specs=[
            pl.BlockSpec((NC, BLK, D), lambda i: (0, i, 0)),
            pl.BlockSpec((BLK, 1), lambda i: (i, 0)),
            pl.BlockSpec((NC, BLK, D), lambda i: (0, i, 0)),
            pl.BlockSpec((BLK, 1), lambda i: (i, 0)),
            pl.BlockSpec((1, D), lambda i: (0, 0)),
        ],
        out_specs=pl.BlockSpec((BLK, D), lambda i: (i, 0)),
        out_shape=jax.ShapeDtypeStruct((N, D), jnp.float32),
    )(acc_p, den, acc2_p, num, bias_vec)


def kernel(ents_embed_input, rels_embed_input, W_ent, W_rel, bias_vec,
           near_rels_num, edge_index, rel_rows, rel_vals):
    m = _embed(ents_embed_input, W_ent)
    rows = edge_index[0].reshape(NW, EPT)
    cols = edge_index[1].reshape(NW, EPT)
    acc_p, den_p = _sc_edges(m, rows, cols)

    pad = REPAD - RE
    dead = N + (jnp.arange(pad, dtype=jnp.int32) % (ACC_ROWS - N))
    rrows = jnp.concatenate([rel_rows, dead]).reshape(NW, REPT)
    rvals = jnp.concatenate(
        [rel_vals, jnp.zeros((pad,), jnp.int32)]).reshape(NW, REPT)
    acc2_p = _sc_rels(rels_embed_input, rrows, rvals)

    den = (den_p[0, :N] + den_p[1, :N]).reshape(N, 1)
    num = near_rels_num.reshape(N, 1)
    return _final(acc_p[:, :N], den, acc2_p[:, :N], num, bias_vec)


# speedup vs baseline: 9.5320x; 1.2361x over previous
import functools

import jax
import jax.numpy as jnp
from jax import lax
from jax.experimental import pallas as pl
from jax.experimental.pallas import tpu as pltpu
from jax.experimental.pallas import tpu_sc as plsc

MIN_NORM = 1e-10
PROJ_EPS = 1e-5
COMBINE_RELS_WEIGHT = 0.1

N = 10000
D = 128
E = 320000
RE = 200000
NC, NS, L = 2, 16, 16
NW = NC * NS

ACC_ROWS = 10240
ROWS_PER_TILE = ACC_ROWS // NS
ZCH = 16

EPT = E // NW
CH = 16
NCH = EPT // CH
NBUF = 4
NBLK = 156

REPAD = NW * 6400
REPT = REPAD // NW
RNCH = REPT // CH
RBUF = 8
RBLK = RNCH // RBUF


def _embed_body(x_ref, w_ref, m_ref):
    x = x_ref[...]
    n = jnp.sqrt(jnp.sum(x * x, axis=1, keepdims=True))
    n = jnp.maximum(n, MIN_NORM)
    n_c = jnp.clip(n, MIN_NORM, 1.0 - PROJ_EPS)
    at = 0.5 * (jnp.log(1.0 + n_c) - jnp.log(1.0 - n_c))
    t = at * x / n
    m_ref[...] = jnp.dot(t, w_ref[...], precision=lax.Precision.HIGHEST,
                         preferred_element_type=jnp.float32)


BLK = 1000


def _embed(ents, w_ent):
    return pl.pallas_call(
        _embed_body,
        grid=(N // BLK,),
        in_specs=[
            pl.BlockSpec((BLK, D), lambda i: (i, 0)),
            pl.BlockSpec((D, D), lambda i: (0, 0)),
        ],
        out_specs=pl.BlockSpec((BLK, D), lambda i: (i, 0)),
        out_shape=jax.ShapeDtypeStruct((N, D), jnp.float32),
    )(ents, w_ent)


def _sc_edges_body(m_hbm, rows_hbm, cols_hbm, acc_out, den_out,
                   idxr1_v, idxc1_v, mr_v, mc_v, w_v, dot_v, zrow_v, zden_v,
                   gs0, gs1, gs2, gs3, ss0, ss1, ss2, ss3, acc_sh, den_sh):
    cid = lax.axis_index("c")
    sid = lax.axis_index("s")
    wid = cid * NS + sid
    zv = jnp.zeros((L,), jnp.float32)
    gsems = [gs0, gs1, gs2, gs3]
    ssems = [ss0, ss1, ss2, ss3]

    def _zrow(i, _):
        for k in range(D // L):
            zrow_v[i, pl.ds(k * L, L)] = zv
        return ()
    lax.fori_loop(0, ZCH, _zrow, ())

    def _zdv(i, _):
        zden_v[pl.ds(i * L, L)] = zv
        return ()
    lax.fori_loop(0, ROWS_PER_TILE // L, _zdv, ())

    def _zacc(j, _):
        pltpu.sync_copy(zrow_v, acc_sh.at[pl.ds(sid * ROWS_PER_TILE + j * ZCH, ZCH)])
        return ()
    lax.fori_loop(0, ROWS_PER_TILE // ZCH, _zacc, ())

    pltpu.sync_copy(zden_v, den_sh.at[pl.ds(sid * ROWS_PER_TILE, ROWS_PER_TILE)])

    pltpu.sync_copy(rows_hbm.at[wid], idxr1_v)
    pltpu.sync_copy(cols_hbm.at[wid], idxc1_v)

    iot = lax.iota(jnp.int32, L)

    def _ridx(i):
        return plsc.load_gather(idxr1_v, [i * CH + iot])

    def _cidx(i):
        return plsc.load_gather(idxc1_v, [i * CH + iot])

    def _fire_gather(i, b):
        pltpu.async_copy(m_hbm.at[_ridx(i)], mr_v.at[b], gsems[b])
        pltpu.async_copy(m_hbm.at[_cidx(i)], mc_v.at[b], gsems[b])

    def _wait_gather(i, b):
        pltpu.make_async_copy(m_hbm.at[iot], mr_v.at[b], gsems[b]).wait()
        pltpu.make_async_copy(m_hbm.at[iot], mc_v.at[b], gsems[b]).wait()

    def _fire_scatter(i, b):
        rv = _ridx(i)
        pltpu.async_copy(mc_v.at[b], acc_sh.at[rv], ssems[b], add=True)
        pltpu.async_copy(w_v.at[b], den_sh.at[rv], ssems[b], add=True)

    def _wait_scatter(i, b):
        pltpu.make_async_copy(mc_v.at[b], acc_sh.at[iot], ssems[b]).wait()
        pltpu.make_async_copy(w_v.at[b], den_sh.at[iot], ssems[b]).wait()

    def _compute(b):
        for j in range(L):
            acc = mr_v[b, j, pl.ds(0, L)] * mc_v[b, j, pl.ds(0, L)]
            for k in range(1, D // L):
                acc = acc + (mr_v[b, j, pl.ds(k * L, L)]
                             * mc_v[b, j, pl.ds(k * L, L)])
            plsc.store_scatter(dot_v, [j * (L + 1) + iot], acc)
        s = plsc.load_gather(dot_v, [iot * (L + 1)])
        for k in range(1, L):
            s = s + plsc.load_gather(dot_v, [iot * (L + 1) + k])
        w16 = jnp.exp(s)
        w_v[b, pl.ds(0, L)] = w16
        for j in range(L):
            wv = jnp.broadcast_to(w16[j], (L,))
            for k in range(D // L):
                mc_v[b, j, pl.ds(k * L, L)] = mc_v[b, j, pl.ds(k * L, L)] * wv

    plsc.subcore_barrier()

    for b in range(NBUF):
        _fire_gather(b, b)

    def _blk(g, _):
        for b in range(NBUF):
            i = g * NBUF + b
            _wait_gather(i, b)

            @pl.when(g >= 1)
            def _():
                _wait_scatter(i - NBUF, b)

            _compute(b)

            @pl.when(g < NBLK - 1)
            def _():
                _fire_gather(i + NBUF, b)

            _fire_scatter(i, b)
        return ()
    lax.fori_loop(0, NBLK, _blk, ())

    for b in range(NBUF):
        _wait_scatter((NBLK - 1) * NBUF + b, b)

    for t in range(NBUF * NBLK, NCH):
        b = t - NBUF * NBLK
        pltpu.async_copy(m_hbm.at[_ridx(t)], mr_v.at[b], gsems[b]).wait()
        pltpu.async_copy(m_hbm.at[_cidx(t)], mc_v.at[b], gsems[b]).wait()
        _compute(b)
        rv = _ridx(t)
        pltpu.async_copy(mc_v.at[b], acc_sh.at[rv], ssems[b], add=True).wait()
        pltpu.async_copy(w_v.at[b], den_sh.at[rv], ssems[b], add=True).wait()

    plsc.subcore_barrier()
    pltpu.sync_copy(acc_sh.at[pl.ds(sid * ROWS_PER_TILE, ROWS_PER_TILE)],
                    acc_out.at[cid, pl.ds(sid * ROWS_PER_TILE, ROWS_PER_TILE)])
    pltpu.sync_copy(den_sh.at[pl.ds(sid * ROWS_PER_TILE, ROWS_PER_TILE)],
                    den_out.at[cid, pl.ds(sid * ROWS_PER_TILE, ROWS_PER_TILE)])


_sc_edges = functools.partial(
    pl.kernel,
    out_type=(jax.ShapeDtypeStruct((NC, ACC_ROWS, D), jnp.float32),
              jax.ShapeDtypeStruct((NC, ACC_ROWS), jnp.float32)),
    mesh=plsc.VectorSubcoreMesh(core_axis_name="c", subcore_axis_name="s"),
    compiler_params=pltpu.CompilerParams(needs_layout_passes=False),
    scratch_types=(
        [
            pltpu.VMEM((EPT,), jnp.int32),
            pltpu.VMEM((EPT,), jnp.int32),
            pltpu.VMEM((NBUF, CH, D), jnp.float32),
            pltpu.VMEM((NBUF, CH, D), jnp.float32),
            pltpu.VMEM((NBUF, CH), jnp.float32),
            pltpu.VMEM((L * (L + 1),), jnp.float32),
            pltpu.VMEM((ZCH, D), jnp.float32),
            pltpu.VMEM((ROWS_PER_TILE,), jnp.float32),
        ]
        + [pltpu.SemaphoreType.DMA] * (2 * NBUF)
        + [
            pltpu.VMEM_SHARED((ACC_ROWS, D), jnp.float32),
            pltpu.VMEM_SHARED((ACC_ROWS,), jnp.float32),
        ]
    ),
)(_sc_edges_body)


NRELS = 500
RSH = 512


def _sc_rels_body(rels_hbm, rrows_hbm, rvals_hbm, acc_out,
                  row1_v, val1_v, emb_v, zrow_v,
                  g0, g1, g2, g3, g4, g5, g6, g7,
                  s0, s1, s2, s3, s4, s5, s6, s7, acc_sh, rels_sh):
    cid = lax.axis_index("c")
    sid = lax.axis_index("s")
    wid = cid * NS + sid
    zv = jnp.zeros((L,), jnp.float32)
    gsems = [g0, g1, g2, g3, g4, g5, g6, g7]
    ssems = [s0, s1, s2, s3, s4, s5, s6, s7]

    def _zrow(i, _):
        for k in range(D // L):
            zrow_v[i, pl.ds(k * L, L)] = zv
        return ()
    lax.fori_loop(0, ZCH, _zrow, ())

    def _zacc(j, _):
        pltpu.sync_copy(zrow_v, acc_sh.at[pl.ds(sid * ROWS_PER_TILE + j * ZCH, ZCH)])
        return ()
    lax.fori_loop(0, ROWS_PER_TILE // ZCH, _zacc, ())

    pltpu.sync_copy(rrows_hbm.at[wid], row1_v)
    pltpu.sync_copy(rvals_hbm.at[wid], val1_v)

    @pl.when(sid < NS - 1)
    def _():
        pltpu.sync_copy(rels_hbm.at[pl.ds(sid * 32, 32)],
                        rels_sh.at[pl.ds(sid * 32, 32)])

    @pl.when(sid == NS - 1)
    def _():
        pltpu.sync_copy(rels_hbm.at[pl.ds(480, NRELS - 480)],
                        rels_sh.at[pl.ds(480, NRELS - 480)])

    iot = lax.iota(jnp.int32, L)

    def _rowi(i):
        return plsc.load_gather(row1_v, [i * CH + iot])

    def _vali(i):
        return plsc.load_gather(val1_v, [i * CH + iot])

    plsc.subcore_barrier()

    def _blk(g, _):
        for b in range(RBUF):
            i = g * RBUF + b

            @pl.when(g >= 1)
            def _():
                pltpu.make_async_copy(emb_v.at[b], acc_sh.at[iot],
                                      ssems[b]).wait()

            pltpu.async_copy(rels_sh.at[_vali(i)], emb_v.at[b], gsems[b])
        for b in range(RBUF):
            i = g * RBUF + b
            pltpu.make_async_copy(rels_sh.at[iot], emb_v.at[b],
                                  gsems[b]).wait()
            pltpu.async_copy(emb_v.at[b], acc_sh.at[_rowi(i)], ssems[b],
                             add=True)
        return ()
    lax.fori_loop(0, RBLK, _blk, ())

    for b in range(RBUF):
        pltpu.make_async_copy(emb_v.at[b], acc_sh.at[iot], ssems[b]).wait()

    plsc.subcore_barrier()
    pltpu.sync_copy(acc_sh.at[pl.ds(sid * ROWS_PER_TILE, ROWS_PER_TILE)],
                    acc_out.at[cid, pl.ds(sid * ROWS_PER_TILE, ROWS_PER_TILE)])


_sc_rels = functools.partial(
    pl.kernel,
    out_type=jax.ShapeDtypeStruct((NC, ACC_ROWS, D), jnp.float32),
    mesh=plsc.VectorSubcoreMesh(core_axis_name="c", subcore_axis_name="s"),
    compiler_params=pltpu.CompilerParams(needs_layout_passes=False),
    scratch_types=(
        [
            pltpu.VMEM((REPT,), jnp.int32),
            pltpu.VMEM((REPT,), jnp.int32),
            pltpu.VMEM((RBUF, CH, D), jnp.float32),
            pltpu.VMEM((ZCH, D), jnp.float32),
        ]
        + [pltpu.SemaphoreType.DMA] * (2 * RBUF)
        + [pltpu.VMEM_SHARED((ACC_ROWS, D), jnp.float32),
           pltpu.VMEM_SHARED((RSH, D), jnp.float32)]
    ),
)(_sc_rels_body)


def _norm(x):
    return jnp.maximum(jnp.sqrt(jnp.sum(x * x, axis=-1, keepdims=True)), MIN_NORM)


def _exp_map_zero(v):
    n = _norm(v)
    return jnp.tanh(n) * v / n


def _proj(x):
    n = _norm(x)
    max_norm = 1.0 - PROJ_EPS
    return jnp.where(n > max_norm, x / n * max_norm, x)


def _mobius(x, y):
    x2 = jnp.sum(x * x, axis=-1, keepdims=True)
    y2 = jnp.sum(y * y, axis=-1, keepdims=True)
    xy = jnp.sum(x * y, axis=-1, keepdims=True)
    num = (1.0 + 2.0 * xy + y2) * x + (1.0 - x2) * y
    den = 1.0 + 2.0 * xy + x2 * y2
    return num / jnp.clip(den, MIN_NORM, None)


def _final_body(acc_ref, den_ref, acc2_ref, num_ref, bias_ref, out_ref):
    acc = acc_ref[0] + acc_ref[1]
    a2 = acc2_ref[0] + acc2_ref[1]
    den = jnp.clip(den_ref[...], 1e-15, None)
    v = acc / den + COMBINE_RELS_WEIGHT * (a2 / num_ref[...])
    out1 = _proj(_exp_map_zero(v))
    b = _proj(_exp_map_zero(bias_ref[...]))
    out_ref[...] = _proj(_mobius(out1, b))


def _final(acc_p, den, acc2_p, num, bias_vec):
    return pl.pallas_call(
        _final_body,
        grid=(N // BLK,),
        in_specs=[
            pl.BlockSpec((NC, BLK, D), lambda i: (0, i, 0)),
            pl.BlockSpec((BLK, 1), lambda i: (i, 0)),
            pl.BlockSpec((NC, BLK, D), lambda i: (0, i, 0)),
            pl.BlockSpec((BLK, 1), lambda i: (i, 0)),
            pl.BlockSpec((1, D), lambda i: (0, 0)),
        ],
        out_specs=pl.BlockSpec((BLK, D), lambda i: (i, 0)),
        out_shape=jax.ShapeDtypeStruct((N, D), jnp.float32),
    )(acc_p, den, acc2_p, num, bias_vec)


def kernel(ents_embed_input, rels_embed_input, W_ent, W_rel, bias_vec,
           near_rels_num, edge_index, rel_rows, rel_vals):
    m = _embed(ents_embed_input, W_ent)
    rows = edge_index[0].reshape(NW, EPT)
    cols = edge_index[1].reshape(NW, EPT)
    acc_p, den_p = _sc_edges(m, rows, cols)

    pad = REPAD - RE
    dead = N + (jnp.arange(pad, dtype=jnp.int32) % (ACC_ROWS - N))
    rrows = jnp.concatenate([rel_rows, dead]).reshape(NW, REPT)
    rvals = jnp.concatenate(
        [rel_vals, jnp.zeros((pad,), jnp.int32)]).reshape(NW, REPT)
    acc2_p = _sc_rels(rels_embed_input, rrows, rvals)

    den = (den_p[0, :N] + den_p[1, :N]).reshape(N, 1)
    num = near_rels_num.reshape(N, 1)
    return _final(acc_p[:, :N], den, acc2_p[:, :N], num, bias_vec)


# speedup vs baseline: 9.8296x; 1.0312x over previous
import functools

import jax
import jax.numpy as jnp
from jax import lax
from jax.experimental import pallas as pl
from jax.experimental.pallas import tpu as pltpu
from jax.experimental.pallas import tpu_sc as plsc

MIN_NORM = 1e-10
PROJ_EPS = 1e-5
COMBINE_RELS_WEIGHT = 0.1

N = 10000
D = 128
E = 320000
RE = 200000
NC, NS, L = 2, 16, 16
NW = NC * NS

ACC_ROWS = 10240
ROWS_PER_TILE = ACC_ROWS // NS
ZCH = 64

EPT = E // NW
CH = 16
NCH = EPT // CH
NBUF = 4
NBLK = 156

REPAD = NW * 6400
REPT = REPAD // NW
RNCH = REPT // CH
RBUF = 8
RBLK = RNCH // RBUF


def _embed_body(x_ref, w_ref, m_ref):
    x = x_ref[...]
    n = jnp.sqrt(jnp.sum(x * x, axis=1, keepdims=True))
    n = jnp.maximum(n, MIN_NORM)
    n_c = jnp.clip(n, MIN_NORM, 1.0 - PROJ_EPS)
    at = 0.5 * (jnp.log(1.0 + n_c) - jnp.log(1.0 - n_c))
    t = at * x / n
    m_ref[...] = jnp.dot(t, w_ref[...], precision=lax.Precision.HIGHEST,
                         preferred_element_type=jnp.float32)


BLK = 1000


def _embed(ents, w_ent):
    return pl.pallas_call(
        _embed_body,
        grid=(N // BLK,),
        in_specs=[
            pl.BlockSpec((BLK, D), lambda i: (i, 0)),
            pl.BlockSpec((D, D), lambda i: (0, 0)),
        ],
        out_specs=pl.BlockSpec((BLK, D), lambda i: (i, 0)),
        out_shape=jax.ShapeDtypeStruct((N, D), jnp.float32),
    )(ents, w_ent)


def _sc_edges_body(m_hbm, rows_hbm, cols_hbm, acc_out, den_out,
                   idxr1_v, idxc1_v, mr_v, mc_v, w_v, dot_v, zrow_v, zden_v,
                   gs0, gs1, gs2, gs3, ss0, ss1, ss2, ss3, acc_sh, den_sh):
    cid = lax.axis_index("c")
    sid = lax.axis_index("s")
    wid = cid * NS + sid
    zv = jnp.zeros((L,), jnp.float32)
    gsems = [gs0, gs1, gs2, gs3]
    ssems = [ss0, ss1, ss2, ss3]

    pltpu.async_copy(rows_hbm.at[wid], idxr1_v, gs0)
    pltpu.async_copy(cols_hbm.at[wid], idxc1_v, gs1)

    def _zrow(i, _):
        for k in range(D // L):
            zrow_v[i, pl.ds(k * L, L)] = zv
        return ()
    lax.fori_loop(0, ZCH, _zrow, ())

    def _zdv(i, _):
        zden_v[pl.ds(i * L, L)] = zv
        return ()
    lax.fori_loop(0, ROWS_PER_TILE // L, _zdv, ())

    pltpu.make_async_copy(rows_hbm.at[wid], idxr1_v, gs0).wait()
    pltpu.make_async_copy(cols_hbm.at[wid], idxc1_v, gs1).wait()

    iot = lax.iota(jnp.int32, L)

    def _ridx(i):
        return plsc.load_gather(idxr1_v, [i * CH + iot])

    def _cidx(i):
        return plsc.load_gather(idxc1_v, [i * CH + iot])

    def _fire_gather(i, b):
        pltpu.async_copy(m_hbm.at[_ridx(i)], mr_v.at[b], gsems[b])
        pltpu.async_copy(m_hbm.at[_cidx(i)], mc_v.at[b], gsems[b])

    def _wait_gather(i, b):
        pltpu.make_async_copy(m_hbm.at[iot], mr_v.at[b], gsems[b]).wait()
        pltpu.make_async_copy(m_hbm.at[iot], mc_v.at[b], gsems[b]).wait()

    def _fire_scatter(i, b):
        rv = _ridx(i)
        pltpu.async_copy(mc_v.at[b], acc_sh.at[rv], ssems[b], add=True)
        pltpu.async_copy(w_v.at[b], den_sh.at[rv], ssems[b], add=True)

    def _wait_scatter(i, b):
        pltpu.make_async_copy(mc_v.at[b], acc_sh.at[iot], ssems[b]).wait()
        pltpu.make_async_copy(w_v.at[b], den_sh.at[iot], ssems[b]).wait()

    def _compute(b):
        for j in range(L):
            acc = mr_v[b, j, pl.ds(0, L)] * mc_v[b, j, pl.ds(0, L)]
            for k in range(1, D // L):
                acc = acc + (mr_v[b, j, pl.ds(k * L, L)]
                             * mc_v[b, j, pl.ds(k * L, L)])
            plsc.store_scatter(dot_v, [j * (L + 1) + iot], acc)
        s = plsc.load_gather(dot_v, [iot * (L + 1)])
        for k in range(1, L):
            s = s + plsc.load_gather(dot_v, [iot * (L + 1) + k])
        w16 = jnp.exp(s)
        w_v[b, pl.ds(0, L)] = w16
        for j in range(L):
            wv = jnp.broadcast_to(w16[j], (L,))
            for k in range(D // L):
                mc_v[b, j, pl.ds(k * L, L)] = mc_v[b, j, pl.ds(k * L, L)] * wv

    for b in range(NBUF):
        _fire_gather(b, b)

    def _zacc(j, _):
        pltpu.sync_copy(zrow_v, acc_sh.at[pl.ds(sid * ROWS_PER_TILE + j * ZCH, ZCH)])
        return ()
    lax.fori_loop(0, ROWS_PER_TILE // ZCH, _zacc, ())

    pltpu.sync_copy(zden_v, den_sh.at[pl.ds(sid * ROWS_PER_TILE, ROWS_PER_TILE)])

    plsc.subcore_barrier()

    def _blk(g, _):
        for b in range(NBUF):
            i = g * NBUF + b
            _wait_gather(i, b)

            @pl.when(g >= 1)
            def _():
                _wait_scatter(i - NBUF, b)

            _compute(b)

            @pl.when(g < NBLK - 1)
            def _():
                _fire_gather(i + NBUF, b)

            _fire_scatter(i, b)
        return ()
    lax.fori_loop(0, NBLK, _blk, ())

    for b in range(NBUF):
        _wait_scatter((NBLK - 1) * NBUF + b, b)

    for t in range(NBUF * NBLK, NCH):
        b = t - NBUF * NBLK
        pltpu.async_copy(m_hbm.at[_ridx(t)], mr_v.at[b], gsems[b]).wait()
        pltpu.async_copy(m_hbm.at[_cidx(t)], mc_v.at[b], gsems[b]).wait()
        _compute(b)
        rv = _ridx(t)
        pltpu.async_copy(mc_v.at[b], acc_sh.at[rv], ssems[b], add=True).wait()
        pltpu.async_copy(w_v.at[b], den_sh.at[rv], ssems[b], add=True).wait()

    plsc.subcore_barrier()
    pltpu.sync_copy(acc_sh.at[pl.ds(sid * ROWS_PER_TILE, ROWS_PER_TILE)],
                    acc_out.at[cid, pl.ds(sid * ROWS_PER_TILE, ROWS_PER_TILE)])
    pltpu.sync_copy(den_sh.at[pl.ds(sid * ROWS_PER_TILE, ROWS_PER_TILE)],
                    den_out.at[cid, pl.ds(sid * ROWS_PER_TILE, ROWS_PER_TILE)])


_sc_edges = functools.partial(
    pl.kernel,
    out_type=(jax.ShapeDtypeStruct((NC, ACC_ROWS, D), jnp.float32),
              jax.ShapeDtypeStruct((NC, ACC_ROWS), jnp.float32)),
    mesh=plsc.VectorSubcoreMesh(core_axis_name="c", subcore_axis_name="s"),
    compiler_params=pltpu.CompilerParams(needs_layout_passes=False),
    scratch_types=(
        [
            pltpu.VMEM((EPT,), jnp.int32),
            pltpu.VMEM((EPT,), jnp.int32),
            pltpu.VMEM((NBUF, CH, D), jnp.float32),
            pltpu.VMEM((NBUF, CH, D), jnp.float32),
            pltpu.VMEM((NBUF, CH), jnp.float32),
            pltpu.VMEM((L * (L + 1),), jnp.float32),
            pltpu.VMEM((ZCH, D), jnp.float32),
            pltpu.VMEM((ROWS_PER_TILE,), jnp.float32),
        ]
        + [pltpu.SemaphoreType.DMA] * (2 * NBUF)
        + [
            pltpu.VMEM_SHARED((ACC_ROWS, D), jnp.float32),
            pltpu.VMEM_SHARED((ACC_ROWS,), jnp.float32),
        ]
    ),
)(_sc_edges_body)


NRELS = 500
RSH = 512


def _sc_rels_body(rels_hbm, rrows_hbm, rvals_hbm, acc_out,
                  row1_v, val1_v, emb_v, zrow_v,
                  g0, g1, g2, g3, g4, g5, g6, g7,
                  s0, s1, s2, s3, s4, s5, s6, s7, acc_sh, rels_sh):
    cid = lax.axis_index("c")
    sid = lax.axis_index("s")
    wid = cid * NS + sid
    zv = jnp.zeros((L,), jnp.float32)
    gsems = [g0, g1, g2, g3, g4, g5, g6, g7]
    ssems = [s0, s1, s2, s3, s4, s5, s6, s7]

    def _zrow(i, _):
        for k in range(D // L):
            zrow_v[i, pl.ds(k * L, L)] = zv
        return ()
    lax.fori_loop(0, ZCH, _zrow, ())

    def _zacc(j, _):
        pltpu.sync_copy(zrow_v, acc_sh.at[pl.ds(sid * ROWS_PER_TILE + j * ZCH, ZCH)])
        return ()
    lax.fori_loop(0, ROWS_PER_TILE // ZCH, _zacc, ())

    pltpu.sync_copy(rrows_hbm.at[wid], row1_v)
    pltpu.sync_copy(rvals_hbm.at[wid], val1_v)

    @pl.when(sid < NS - 1)
    def _():
        pltpu.sync_copy(rels_hbm.at[pl.ds(sid * 32, 32)],
                        rels_sh.at[pl.ds(sid * 32, 32)])

    @pl.when(sid == NS - 1)
    def _():
        pltpu.sync_copy(rels_hbm.at[pl.ds(480, NRELS - 480)],
                        rels_sh.at[pl.ds(480, NRELS - 480)])

    iot = lax.iota(jnp.int32, L)

    def _rowi(i):
        return plsc.load_gather(row1_v, [i * CH + iot])

    def _vali(i):
        return plsc.load_gather(val1_v, [i * CH + iot])

    plsc.subcore_barrier()

    def _blk(g, _):
        for b in range(RBUF):
            i = g * RBUF + b

            @pl.when(g >= 1)
            def _():
                pltpu.make_async_copy(emb_v.at[b], acc_sh.at[iot],
                                      ssems[b]).wait()

            pltpu.async_copy(rels_sh.at[_vali(i)], emb_v.at[b], gsems[b])
        for b in range(RBUF):
            i = g * RBUF + b
            pltpu.make_async_copy(rels_sh.at[iot], emb_v.at[b],
                                  gsems[b]).wait()
            pltpu.async_copy(emb_v.at[b], acc_sh.at[_rowi(i)], ssems[b],
                             add=True)
        return ()
    lax.fori_loop(0, RBLK, _blk, ())

    for b in range(RBUF):
        pltpu.make_async_copy(emb_v.at[b], acc_sh.at[iot], ssems[b]).wait()

    plsc.subcore_barrier()
    pltpu.sync_copy(acc_sh.at[pl.ds(sid * ROWS_PER_TILE, ROWS_PER_TILE)],
                    acc_out.at[cid, pl.ds(sid * ROWS_PER_TILE, ROWS_PER_TILE)])


_sc_rels = functools.partial(
    pl.kernel,
    out_type=jax.ShapeDtypeStruct((NC, ACC_ROWS, D), jnp.float32),
    mesh=plsc.VectorSubcoreMesh(core_axis_name="c", subcore_axis_name="s"),
    compiler_params=pltpu.CompilerParams(needs_layout_passes=False),
    scratch_types=(
        [
            pltpu.VMEM((REPT,), jnp.int32),
            pltpu.VMEM((REPT,), jnp.int32),
            pltpu.VMEM((RBUF, CH, D), jnp.float32),
            pltpu.VMEM((ZCH, D), jnp.float32),
        ]
        + [pltpu.SemaphoreType.DMA] * (2 * RBUF)
        + [pltpu.VMEM_SHARED((ACC_ROWS, D), jnp.float32),
           pltpu.VMEM_SHARED((RSH, D), jnp.float32)]
    ),
)(_sc_rels_body)


def _norm(x):
    return jnp.maximum(jnp.sqrt(jnp.sum(x * x, axis=-1, keepdims=True)), MIN_NORM)


def _exp_map_zero(v):
    n = _norm(v)
    return jnp.tanh(n) * v / n


def _proj(x):
    n = _norm(x)
    max_norm = 1.0 - PROJ_EPS
    return jnp.where(n > max_norm, x / n * max_norm, x)


def _mobius(x, y):
    x2 = jnp.sum(x * x, axis=-1, keepdims=True)
    y2 = jnp.sum(y * y, axis=-1, keepdims=True)
    xy = jnp.sum(x * y, axis=-1, keepdims=True)
    num = (1.0 + 2.0 * xy + y2) * x + (1.0 - x2) * y
    den = 1.0 + 2.0 * xy + x2 * y2
    return num / jnp.clip(den, MIN_NORM, None)


def _final_body(acc_ref, den_ref, acc2_ref, num_ref, bias_ref, out_ref):
    acc = acc_ref[0] + acc_ref[1]
    a2 = acc2_ref[0] + acc2_ref[1]
    den = jnp.clip(den_ref[...], 1e-15, None)
    v = acc / den + COMBINE_RELS_WEIGHT * (a2 / num_ref[...])
    out1 = _proj(_exp_map_zero(v))
    b = _proj(_exp_map_zero(bias_ref[...]))
    out_ref[...] = _proj(_mobius(out1, b))


def _final(acc_p, den, acc2_p, num, bias_vec):
    return pl.pallas_call(
        _final_body,
        grid=(N // BLK,),
        in_specs=[
            pl.BlockSpec((NC, BLK, D), lambda i: (0, i, 0)),
            pl.BlockSpec((BLK, 1), lambda i: (i, 0)),
            pl.BlockSpec((NC, BLK, D), lambda i: (0, i, 0)),
            pl.BlockSpec((BLK, 1), lambda i: (i, 0)),
            pl.BlockSpec((1, D), lambda i: (0, 0)),
        ],
        out_specs=pl.BlockSpec((BLK, D), lambda i: (i, 0)),
        out_shape=jax.ShapeDtypeStruct((N, D), jnp.float32),
    )(acc_p, den, acc2_p, num, bias_vec)


def kernel(ents_embed_input, rels_embed_input, W_ent, W_rel, bias_vec,
           near_rels_num, edge_index, rel_rows, rel_vals):
    m = _embed(ents_embed_input, W_ent)
    rows = edge_index[0].reshape(NW, EPT)
    cols = edge_index[1].reshape(NW, EPT)
    acc_p, den_p = _sc_edges(m, rows, cols)

    pad = REPAD - RE
    dead = N + (jnp.arange(pad, dtype=jnp.int32) % (ACC_ROWS - N))
    rrows = jnp.concatenate([rel_rows, dead]).reshape(NW, REPT)
    rvals = jnp.concatenate(
        [rel_vals, jnp.zeros((pad,), jnp.int32)]).reshape(NW, REPT)
    acc2_p = _sc_rels(rels_embed_input, rrows, rvals)

    den = (den_p[0, :N] + den_p[1, :N]).reshape(N, 1)
    num = near_rels_num.reshape(N, 1)
    return _final(acc_p[:, :N], den, acc2_p[:, :N], num, bias_vec)
